# baseline (speedup 1.0000x reference)
import functools

import jax
import jax.numpy as jnp
from jax import lax
from jax.experimental import pallas as pl
from jax.experimental.pallas import tpu as pltpu
from jax.experimental.pallas import tpu_sc as plsc

N = 10000
E = 320000
D_FEAT = 128
ENV_DIM = 128
HIDDEN = 64
NUM_LAYERS = 3
HIST_DIM = 12
NUM_GRAPHS = 64
OUT_DIM = 40
TAU = 0.5

_NC, _NS = 2, 16
_NW = _NC * _NS
_C = 128
_NCHUNK = 80
_EPW = _C * _NCHUNK
_EPAD = _EPW * _NW
_DT = 128
_NROW = 10240
_RPT = _NROW // _NS
_HR = _NROW // 128

_R = 512
_NB = _NROW // _R


def _sc_pass_body(table, u, v, out,
                  uall, v0, v1, rows0, rows1, acc,
                  sg0, sg1, ss0, ss1, si0, si1):
    cid = lax.axis_index("c")
    sid = lax.axis_index("s")
    wid = cid * _NS + sid
    base = wid * _EPW

    zero16 = jnp.zeros((16,), jnp.float32)

    def _zr(i, carry):
        for j in range(_DT // 16):
            rows0[i, pl.ds(j * 16, 16)] = zero16
        return carry
    lax.fori_loop(0, _C, _zr, 0)

    pltpu.sync_copy(u.at[pl.ds(base, _EPW)], uall)
    for k in range(_RPT // _C):
        pltpu.sync_copy(rows0, acc.at[pl.ds(sid * _RPT + k * _C, _C)])

    pltpu.async_copy(v.at[pl.ds(base, _C)], v0, si0)
    pltpu.async_copy(table.at[uall.at[pl.ds(0, _C)]], rows0, sg0)
    plsc.subcore_barrier()

    bufs = ((v0, rows0, sg0, ss0, si0), (v1, rows1, sg1, ss1, si1))

    def _drain_rows(sem, rbuf):
        pltpu.make_async_copy(table.at[pl.ds(0, _C)], rbuf, sem).wait()

    def _pair(gg, carry):
        for b in range(2):
            c = gg * 2 + b
            vc, rc, sgc, ssc, sic = bufs[b]
            vn, rn, sgn, ssn, sin = bufs[1 - b]

            @pl.when(c >= 1)
            def _():
                _drain_rows(ssn, rn)

            @pl.when(c + 1 < _NCHUNK)
            def _():
                pltpu.async_copy(v.at[pl.ds(base + (c + 1) * _C, _C)],
                                 vn, sin)
                pltpu.async_copy(
                    table.at[uall.at[pl.ds((c + 1) * _C, _C)]], rn, sgn)

            pltpu.make_async_copy(v.at[pl.ds(0, _C)], vc, sic).wait()
            _drain_rows(sgc, rc)
            pltpu.async_copy(rc, acc.at[vc], ssc, add=True)
        return carry
    lax.fori_loop(0, _NCHUNK // 2, _pair, 0)

    _drain_rows(ss1, rows1)
    plsc.subcore_barrier()

    pltpu.sync_copy(acc.at[pl.ds(sid * _RPT, _RPT)],
                    out.at[cid, pl.ds(sid * _RPT, _RPT)])


@functools.cache
def _get_sc_pass():
    return pl.kernel(
        _sc_pass_body,
        out_type=jax.ShapeDtypeStruct((_NC, _NROW, _DT), jnp.float32),
        mesh=plsc.VectorSubcoreMesh(core_axis_name="c", subcore_axis_name="s",
                                    num_cores=_NC, num_subcores=_NS),
        compiler_params=pltpu.CompilerParams(needs_layout_passes=False),
        scratch_types=[
            pltpu.VMEM((_EPW,), jnp.int32),
            pltpu.VMEM((_C,), jnp.int32),
            pltpu.VMEM((_C,), jnp.int32),
            pltpu.VMEM((_C, _DT), jnp.float32),
            pltpu.VMEM((_C, _DT), jnp.float32),
            pltpu.VMEM_SHARED((_NROW, _DT), jnp.float32),
            pltpu.SemaphoreType.DMA,
            pltpu.SemaphoreType.DMA,
            pltpu.SemaphoreType.DMA,
            pltpu.SemaphoreType.DMA,
            pltpu.SemaphoreType.DMA,
            pltpu.SemaphoreType.DMA,
        ],
    )


def _sc_pass(table, u, v):
    return _get_sc_pass()(table, u, v)


def _sc_hist_body(kvec, u, v, out_h, kv_v, u_all, v_all, hist, idx_h, acc_h):
    cid = lax.axis_index("c")
    sid = lax.axis_index("s")
    wid = cid * _NS + sid

    zero16 = jnp.zeros((16,), jnp.float32)

    def _zh(i, carry):
        for j in range(128 // 16):
            hist[i, pl.ds(j * 16, 16)] = zero16
        return carry
    lax.fori_loop(0, _HR, _zh, 0)

    iota16 = lax.iota(jnp.int32, 16)
    for j in range(_HR // 16):
        idx_h[pl.ds(j * 16, 16)] = iota16 + j * 16

    @pl.when(sid < _HR // 8)
    def _():
        pltpu.sync_copy(hist.at[pl.ds(0, 8)], acc_h.at[pl.ds(sid * 8, 8)])

    pltpu.sync_copy(kvec, kv_v)
    pltpu.sync_copy(u.at[pl.ds(wid * _EPW, _EPW)], u_all)
    pltpu.sync_copy(v.at[pl.ds(wid * _EPW, _EPW)], v_all)
    plsc.subcore_barrier()

    def _body(i, carry):
        u16 = u_all[pl.ds(i * 16, 16)]
        v16 = v_all[pl.ds(i * 16, 16)]
        kv16 = plsc.load_gather(kv_v, [u16])
        plsc.addupdate_scatter(hist, [v16 >> 7, v16 & 127], kv16)
        return carry
    lax.fori_loop(0, _EPW // 16, _body, 0)

    pltpu.sync_copy(hist, acc_h.at[idx_h], add=True)
    plsc.subcore_barrier()

    @pl.when(sid < _HR // 8)
    def _():
        pltpu.sync_copy(acc_h.at[pl.ds(sid * 8, 8)],
                        out_h.at[cid, pl.ds(sid * 8, 8)])


@functools.cache
def _get_sc_hist():
    return pl.kernel(
        _sc_hist_body,
        out_type=jax.ShapeDtypeStruct((_NC, _HR, 128), jnp.float32),
        mesh=plsc.VectorSubcoreMesh(core_axis_name="c", subcore_axis_name="s",
                                    num_cores=_NC, num_subcores=_NS),
        compiler_params=pltpu.CompilerParams(needs_layout_passes=False),
        scratch_types=[
            pltpu.VMEM((_NROW,), jnp.float32),
            pltpu.VMEM((_EPW,), jnp.int32),
            pltpu.VMEM((_EPW,), jnp.int32),
            pltpu.VMEM((_HR, 128), jnp.float32),
            pltpu.VMEM((_HR,), jnp.int32),
            pltpu.VMEM_SHARED((_HR, 128), jnp.float32),
        ],
    )


def _sc_hist(kvec, u, v):
    return _get_sc_hist()(kvec, u, v)


def _row_spec(w):
    return pl.BlockSpec((_R, w), lambda i: (i, 0))


def _full_spec(shape):
    nd = len(shape)
    return pl.BlockSpec(shape, lambda i, _n=nd: (0,) * _n)


def _ln(hb, g, b):
    mu = jnp.mean(hb, axis=-1, keepdims=True)
    var = jnp.mean((hb - mu) ** 2, axis=-1, keepdims=True)
    return (hb - mu) / jnp.sqrt(var + 1e-5) * g + b


def _enc_body(x_ref, w_ref, b_ref, h_ref):
    h_ref[...] = jax.nn.relu(jnp.dot(x_ref[...], w_ref[...]) + b_ref[...])


def _enc(x, W_in, b_in):
    return pl.pallas_call(
        _enc_body,
        grid=(_NB,),
        in_specs=[_row_spec(D_FEAT), _full_spec((D_FEAT, ENV_DIM)),
                  _full_spec((1, ENV_DIM))],
        out_specs=_row_spec(ENV_DIM),
        out_shape=jax.ShapeDtypeStruct((N, ENV_DIM), jnp.float32),
    )(x, W_in, b_in.reshape(1, ENV_DIM))


def _build_a_body(h_ref, hist_ref, g_ref, b_ref, w1_ref, b1_ref,
                  hn_ref, a_ref):
    hn = _ln(h_ref[...], g_ref[...], b_ref[...])
    hn_ref[...] = hn
    w1 = w1_ref[...]
    z = jnp.dot(hn, w1[:ENV_DIM]) + jnp.dot(hist_ref[...], w1[ENV_DIM:])
    a_ref[...] = jax.nn.relu(z + b1_ref[...])


def _build_a(h, hist, ln_g, ln_b, Wa1, b1):
    return pl.pallas_call(
        _build_a_body,
        grid=(_NB,),
        in_specs=[_row_spec(ENV_DIM), _row_spec(HIST_DIM),
                  _full_spec((1, ENV_DIM)), _full_spec((1, ENV_DIM)),
                  _full_spec((ENV_DIM + HIST_DIM, 2 * HIDDEN)),
                  _full_spec((1, 2 * HIDDEN))],
        out_specs=[_row_spec(ENV_DIM), _row_spec(_DT)],
        out_shape=[jax.ShapeDtypeStruct((N, ENV_DIM), jnp.float32),
                   jax.ShapeDtypeStruct((_NROW, _DT), jnp.float32)],
    )(h, hist, ln_g.reshape(1, -1), ln_b.reshape(1, -1), Wa1,
      b1.reshape(1, -1))


def _keep(agg, W2, b2, un):
    logits = jnp.dot(agg, W2) + b2
    g = -jnp.log(-jnp.log(un))
    t = (logits + g) / TAU
    m = jnp.max(t, axis=-1, keepdims=True)
    e = jnp.exp(t - m)
    y = e / jnp.sum(e, axis=-1, keepdims=True)
    return jnp.where(y[:, 0:1] >= y[:, 1:2], 1.0, 0.0)


def _act_post_body(p0_ref, p1_ref, d0_ref, d1_ref, uni_ref, uno_ref,
                   wi2_ref, bi2_ref, wo2_ref, bo2_ref, hn_ref, wc_ref,
                   bc_ref, c_ref, act_ref, ko_ref):
    p = p0_ref[...] + p1_ref[...]
    deg = jnp.maximum(d0_ref[...] + d1_ref[...], 1.0)
    ki = _keep(p[:, :HIDDEN] / deg, wi2_ref[...], bi2_ref[...], uni_ref[...])
    ko = _keep(p[:, HIDDEN:ENV_DIM] / deg, wo2_ref[...], bo2_ref[...],
               uno_ref[...])
    m = jnp.dot(hn_ref[...], wc_ref[...]) + bc_ref[...]
    c_ref[...] = ko * m
    ko_ref[...] = ko
    act_ref[...] = jnp.concatenate(
        [ki * ko, ki * (1.0 - ko), (1.0 - ki) * ko, (1.0 - ki) * (1.0 - ko)],
        axis=1)


def _act_post(p0, p1, d0, d1, un_in, un_out, Wa_in2, ba_in2, Wa_out2,
              ba_out2, hn, Wc, bc):
    return pl.pallas_call(
        _act_post_body,
        grid=(_NB,),
        in_specs=[_row_spec(_DT), _row_spec(_DT), _row_spec(1), _row_spec(1),
                  _row_spec(2), _row_spec(2),
                  _full_spec((HIDDEN, 2)), _full_spec((1, 2)),
                  _full_spec((HIDDEN, 2)), _full_spec((1, 2)),
                  _row_spec(ENV_DIM), _full_spec((ENV_DIM, ENV_DIM)),
                  _full_spec((1, ENV_DIM))],
        out_specs=[_row_spec(_DT), _row_spec(4), _row_spec(1)],
        out_shape=[jax.ShapeDtypeStruct((_NROW, _DT), jnp.float32),
                   jax.ShapeDtypeStruct((N, 4), jnp.float32),
                   jax.ShapeDtypeStruct((_NROW, 1), jnp.float32)],
    )(p0, p1, d0, d1, un_in, un_out, Wa_in2, ba_in2.reshape(1, -1), Wa_out2,
      ba_out2.reshape(1, -1), hn, Wc, bc.reshape(1, -1))


def _conv_post_body(s0_ref, s1_ref, w0_ref, w1_ref, hn_ref, act_ref, h_ref):
    s = s0_ref[...] + s1_ref[...]
    wdeg = jnp.maximum(w0_ref[...] + w1_ref[...], 1e-6)
    ki = act_ref[:, 0:1] + act_ref[:, 1:2]
    h_ref[...] = hn_ref[...] + ki * jax.nn.relu(s / wdeg)


def _conv_post(s0, s1, w0, w1, hn, act):
    return pl.pallas_call(
        _conv_post_body,
        grid=(_NB,),
        in_specs=[_row_spec(_DT), _row_spec(_DT), _row_spec(1), _row_spec(1),
                  _row_spec(ENV_DIM), _row_spec(4)],
        out_specs=_row_spec(ENV_DIM),
        out_shape=jax.ShapeDtypeStruct((N, ENV_DIM), jnp.float32),
    )(s0, s1, w0, w1, hn, act)


def _final_body(h_ref, g_ref, b_ref, batch_ref, wf_ref, bf_ref,
                pool_ref, res_ref):
    i = pl.program_id(0)

    @pl.when(i == 0)
    def _():
        pool_ref[...] = jnp.zeros((NUM_GRAPHS, ENV_DIM + 16), jnp.float32)

    rows = i * _R + lax.broadcasted_iota(jnp.int32, (_R, 1), 0)
    valid = rows < N
    hf = _ln(h_ref[...], g_ref[...], b_ref[...])
    hf = jnp.where(valid, hf, 0.0)
    gids = lax.broadcasted_iota(jnp.int32, (_R, NUM_GRAPHS), 1)
    onehot = jnp.where((batch_ref[...] == gids) & valid, 1.0, 0.0)
    hext = jnp.concatenate(
        [hf, valid.astype(jnp.float32), jnp.zeros((_R, 15), jnp.float32)],
        axis=1)
    pool_ref[...] += lax.dot_general(onehot, hext, (((0,), (0,)), ((), ())))

    @pl.when(i == _NB - 1)
    def _():
        p = pool_ref[...]
        pooled = p[:, :ENV_DIM] / jnp.maximum(p[:, ENV_DIM:ENV_DIM + 1], 1.0)
        res_ref[...] = jnp.dot(pooled, wf_ref[...]) + bf_ref[...]


def _final(h, ln_g, ln_b, batch2d, W_fin, b_fin):
    pooled, res = pl.pallas_call(
        _final_body,
        grid=(_NB,),
        in_specs=[_row_spec(ENV_DIM), _full_spec((1, ENV_DIM)),
                  _full_spec((1, ENV_DIM)), _row_spec(1),
                  _full_spec((ENV_DIM, OUT_DIM)), _full_spec((1, OUT_DIM))],
        out_specs=[_full_spec((NUM_GRAPHS, ENV_DIM + 16)),
                   _full_spec((NUM_GRAPHS, OUT_DIM))],
        out_shape=[jax.ShapeDtypeStruct((NUM_GRAPHS, ENV_DIM + 16),
                                        jnp.float32),
                   jax.ShapeDtypeStruct((NUM_GRAPHS, OUT_DIM), jnp.float32)],
    )(h, ln_g.reshape(1, -1), ln_b.reshape(1, -1), batch2d, W_fin,
      b_fin.reshape(1, -1))
    return res


def kernel(x, edge_index, pestat, batch, W_in, b_in, ln_g, ln_b, W_conv,
           b_conv, Wa_in1, ba_in1, Wa_in2, ba_in2, Wa_out1, ba_out1,
           Wa_out2, ba_out2, W_fin, b_fin):
    del pestat
    pad = jnp.full((_EPAD - E,), N, jnp.int32)
    u_p = jnp.concatenate([edge_index[0], pad])
    v_p = jnp.concatenate([edge_index[1], pad])
    Wa1 = jnp.concatenate([Wa_in1, Wa_out1], axis=1)
    b1 = jnp.concatenate([ba_in1, ba_out1])
    ones_k = jnp.ones((_NROW,), jnp.float32)

    uns = []
    for i in range(NUM_LAYERS):
        k = jax.random.fold_in(jax.random.key(42), i)
        uns.append(tuple(
            jax.random.uniform(jax.random.fold_in(k, j), (N, 2),
                               minval=1e-6, maxval=1.0 - 1e-6)
            for j in range(2)))

    h = _enc(x, W_in, b_in)
    pd = _sc_hist(ones_k, u_p, v_p)
    d0 = pd[0].reshape(_NROW, 1)
    d1 = pd[1].reshape(_NROW, 1)
    hist = jnp.zeros((N, HIST_DIM), jnp.float32)
    acts = []
    for i in range(NUM_LAYERS):
        hn, A = _build_a(h, hist, ln_g, ln_b, Wa1, b1)
        pz = _sc_pass(A, u_p, v_p)
        ctab, act, ko = _act_post(
            pz[0], pz[1], d0, d1,
            uns[i][0], uns[i][1], Wa_in2, ba_in2, Wa_out2, ba_out2, hn,
            W_conv[i], b_conv[i])
        s = _sc_pass(ctab, u_p, v_p)
        sw = _sc_hist(ko.reshape(_NROW), u_p, v_p)
        h = _conv_post(s[0], s[1], sw[0].reshape(_NROW, 1),
                       sw[1].reshape(_NROW, 1), hn, act)
        acts.append(act)
        if i < NUM_LAYERS - 1:
            hist = jnp.concatenate([hist[:, 4:], act], axis=1)

    result = _final(h, ln_g, ln_b, batch.reshape(N, 1), W_fin, b_fin)
    history = jnp.concatenate(
        [jnp.zeros((N, 4), x.dtype), acts[0], acts[1]], axis=1)
    return (result, -jnp.ones((NUM_LAYERS,), x.dtype), history)


# speedup vs baseline: 2.9340x; 2.9340x over previous
import functools

import jax
import jax.numpy as jnp
from jax import lax
from jax.experimental import pallas as pl
from jax.experimental.pallas import tpu as pltpu
from jax.experimental.pallas import tpu_sc as plsc

N = 10000
E = 320000
D_FEAT = 128
ENV_DIM = 128
HIDDEN = 64
NUM_LAYERS = 3
HIST_DIM = 12
NUM_GRAPHS = 64
OUT_DIM = 40
TAU = 0.5

_NC, _NS = 2, 16
_NW = _NC * _NS
_C = 128
_NCHUNK = 80
_EPW = _C * _NCHUNK
_EPAD = _EPW * _NW
_DT = 128
_NROW = 10240
_RPT = _NROW // _NS
_HR = _NROW // 128

_R = 512
_NB = _NROW // _R


def _sc_pass_body(table, u, v, out,
                  uall, v0, v1, rows0, rows1, acc,
                  sg0, sg1, ss0, ss1, si0, si1):
    cid = lax.axis_index("c")
    sid = lax.axis_index("s")
    wid = cid * _NS + sid
    base = wid * _EPW

    zero16 = jnp.zeros((16,), jnp.float32)

    def _zr(i, carry):
        for j in range(_DT // 16):
            rows0[i, pl.ds(j * 16, 16)] = zero16
        return carry
    lax.fori_loop(0, _C, _zr, 0)

    pltpu.sync_copy(u.at[pl.ds(base, _EPW)], uall)
    for k in range(_RPT // _C):
        pltpu.sync_copy(rows0, acc.at[pl.ds(sid * _RPT + k * _C, _C)])

    pltpu.async_copy(v.at[pl.ds(base, _C)], v0, si0)
    pltpu.async_copy(table.at[uall.at[pl.ds(0, _C)]], rows0, sg0)
    plsc.subcore_barrier()

    bufs = ((v0, rows0, sg0, ss0, si0), (v1, rows1, sg1, ss1, si1))

    def _drain_rows(sem, rbuf):
        pltpu.make_async_copy(table.at[pl.ds(0, _C)], rbuf, sem).wait()

    def _pair(gg, carry):
        for b in range(2):
            c = gg * 2 + b
            vc, rc, sgc, ssc, sic = bufs[b]
            vn, rn, sgn, ssn, sin = bufs[1 - b]

            @pl.when(c >= 1)
            def _():
                _drain_rows(ssn, rn)

            @pl.when(c + 1 < _NCHUNK)
            def _():
                pltpu.async_copy(v.at[pl.ds(base + (c + 1) * _C, _C)],
                                 vn, sin)
                pltpu.async_copy(
                    table.at[uall.at[pl.ds((c + 1) * _C, _C)]], rn, sgn)

            pltpu.make_async_copy(v.at[pl.ds(0, _C)], vc, sic).wait()
            _drain_rows(sgc, rc)
            pltpu.async_copy(rc, acc.at[vc], ssc, add=True)
        return carry
    lax.fori_loop(0, _NCHUNK // 2, _pair, 0)

    _drain_rows(ss1, rows1)
    plsc.subcore_barrier()

    pltpu.sync_copy(acc.at[pl.ds(sid * _RPT, _RPT)],
                    out.at[cid, pl.ds(sid * _RPT, _RPT)])


@functools.cache
def _get_sc_pass():
    return pl.kernel(
        _sc_pass_body,
        out_type=jax.ShapeDtypeStruct((_NC, _NROW, _DT), jnp.float32),
        mesh=plsc.VectorSubcoreMesh(core_axis_name="c", subcore_axis_name="s",
                                    num_cores=_NC, num_subcores=_NS),
        compiler_params=pltpu.CompilerParams(needs_layout_passes=False),
        scratch_types=[
            pltpu.VMEM((_EPW,), jnp.int32),
            pltpu.VMEM((_C,), jnp.int32),
            pltpu.VMEM((_C,), jnp.int32),
            pltpu.VMEM((_C, _DT), jnp.float32),
            pltpu.VMEM((_C, _DT), jnp.float32),
            pltpu.VMEM_SHARED((_NROW, _DT), jnp.float32),
            pltpu.SemaphoreType.DMA,
            pltpu.SemaphoreType.DMA,
            pltpu.SemaphoreType.DMA,
            pltpu.SemaphoreType.DMA,
            pltpu.SemaphoreType.DMA,
            pltpu.SemaphoreType.DMA,
        ],
    )


def _sc_pass(table, u, v):
    return _get_sc_pass()(table, u, v)


def _sc_hist_body(kvec, u, v, out_h, kv_v, u_all, v_all, hist, idx_h, acc_h):
    cid = lax.axis_index("c")
    sid = lax.axis_index("s")
    wid = cid * _NS + sid

    zero16 = jnp.zeros((16,), jnp.float32)

    def _zh(i, carry):
        for j in range(128 // 16):
            hist[i, pl.ds(j * 16, 16)] = zero16
        return carry
    lax.fori_loop(0, _HR, _zh, 0)

    iota16 = lax.iota(jnp.int32, 16)
    for j in range(_HR // 16):
        idx_h[pl.ds(j * 16, 16)] = iota16 + j * 16

    @pl.when(sid < _HR // 8)
    def _():
        pltpu.sync_copy(hist.at[pl.ds(0, 8)], acc_h.at[pl.ds(sid * 8, 8)])

    pltpu.sync_copy(kvec, kv_v)
    pltpu.sync_copy(u.at[pl.ds(wid * _EPW, _EPW)], u_all)
    pltpu.sync_copy(v.at[pl.ds(wid * _EPW, _EPW)], v_all)
    plsc.subcore_barrier()

    def _body(i, carry):
        u16 = u_all[pl.ds(i * 16, 16)]
        v16 = v_all[pl.ds(i * 16, 16)]
        kv16 = plsc.load_gather(kv_v, [u16])
        plsc.addupdate_scatter(hist, [v16 >> 7, v16 & 127], kv16)
        return carry
    lax.fori_loop(0, _EPW // 16, _body, 0)

    pltpu.sync_copy(hist, acc_h.at[idx_h], add=True)
    plsc.subcore_barrier()

    @pl.when(sid < _HR // 8)
    def _():
        pltpu.sync_copy(acc_h.at[pl.ds(sid * 8, 8)],
                        out_h.at[cid, pl.ds(sid * 8, 8)])


@functools.cache
def _get_sc_hist():
    return pl.kernel(
        _sc_hist_body,
        out_type=jax.ShapeDtypeStruct((_NC, _HR, 128), jnp.float32),
        mesh=plsc.VectorSubcoreMesh(core_axis_name="c", subcore_axis_name="s",
                                    num_cores=_NC, num_subcores=_NS),
        compiler_params=pltpu.CompilerParams(needs_layout_passes=False),
        scratch_types=[
            pltpu.VMEM((_NROW,), jnp.float32),
            pltpu.VMEM((_EPW,), jnp.int32),
            pltpu.VMEM((_EPW,), jnp.int32),
            pltpu.VMEM((_HR, 128), jnp.float32),
            pltpu.VMEM((_HR,), jnp.int32),
            pltpu.VMEM_SHARED((_HR, 128), jnp.float32),
        ],
    )


def _sc_hist(kvec, u, v):
    return _get_sc_hist()(kvec, u, v)


def _row_spec(w):
    return pl.BlockSpec((_R, w), lambda i: (i, 0))


def _full_spec(shape):
    nd = len(shape)
    return pl.BlockSpec(shape, lambda i, _n=nd: (0,) * _n)


def _ln(hb, g, b):
    mu = jnp.mean(hb, axis=-1, keepdims=True)
    var = jnp.mean((hb - mu) ** 2, axis=-1, keepdims=True)
    return (hb - mu) / jnp.sqrt(var + 1e-5) * g + b


def _enc_body(x_ref, w_ref, b_ref, h_ref):
    h_ref[...] = jax.nn.relu(jnp.dot(x_ref[...], w_ref[...]) + b_ref[...])


def _enc(x, W_in, b_in):
    return pl.pallas_call(
        _enc_body,
        grid=(_NB,),
        in_specs=[_row_spec(D_FEAT), _full_spec((D_FEAT, ENV_DIM)),
                  _full_spec((1, ENV_DIM))],
        out_specs=_row_spec(ENV_DIM),
        out_shape=jax.ShapeDtypeStruct((N, ENV_DIM), jnp.float32),
    )(x, W_in, b_in.reshape(1, ENV_DIM))


def _build_a_body(h_ref, hist_ref, g_ref, b_ref, w1_ref, b1_ref,
                  hn_ref, a_ref):
    hn = _ln(h_ref[...], g_ref[...], b_ref[...])
    hn_ref[...] = hn
    w1 = w1_ref[...]
    z = jnp.dot(hn, w1[:ENV_DIM]) + jnp.dot(hist_ref[...], w1[ENV_DIM:])
    a_ref[...] = jax.nn.relu(z + b1_ref[...])


def _build_a(h, hist, ln_g, ln_b, Wa1, b1):
    return pl.pallas_call(
        _build_a_body,
        grid=(_NB,),
        in_specs=[_row_spec(ENV_DIM), _row_spec(HIST_DIM),
                  _full_spec((1, ENV_DIM)), _full_spec((1, ENV_DIM)),
                  _full_spec((ENV_DIM + HIST_DIM, 2 * HIDDEN)),
                  _full_spec((1, 2 * HIDDEN))],
        out_specs=[_row_spec(ENV_DIM), _row_spec(_DT)],
        out_shape=[jax.ShapeDtypeStruct((N, ENV_DIM), jnp.float32),
                   jax.ShapeDtypeStruct((_NROW, _DT), jnp.float32)],
    )(h, hist, ln_g.reshape(1, -1), ln_b.reshape(1, -1), Wa1,
      b1.reshape(1, -1))


def _keep(agg, W2, b2, un):
    logits = jnp.dot(agg, W2) + b2
    g = -jnp.log(-jnp.log(un))
    t = (logits + g) / TAU
    m = jnp.max(t, axis=-1, keepdims=True)
    e = jnp.exp(t - m)
    y = e / jnp.sum(e, axis=-1, keepdims=True)
    return jnp.where(y[:, 0:1] >= y[:, 1:2], 1.0, 0.0)


def _act_post_body(p0_ref, p1_ref, d0_ref, d1_ref, uni_ref, uno_ref,
                   wi2_ref, bi2_ref, wo2_ref, bo2_ref, hn_ref, wc_ref,
                   bc_ref, c_ref, act_ref, ko_ref):
    p = p0_ref[...] + p1_ref[...]
    deg = jnp.maximum(d0_ref[...] + d1_ref[...], 1.0)
    ki = _keep(p[:, :HIDDEN] / deg, wi2_ref[...], bi2_ref[...], uni_ref[...])
    ko = _keep(p[:, HIDDEN:ENV_DIM] / deg, wo2_ref[...], bo2_ref[...],
               uno_ref[...])
    m = jnp.dot(hn_ref[...], wc_ref[...]) + bc_ref[...]
    c_ref[...] = ko * m
    ko_ref[...] = ko
    act_ref[...] = jnp.concatenate(
        [ki * ko, ki * (1.0 - ko), (1.0 - ki) * ko, (1.0 - ki) * (1.0 - ko)],
        axis=1)


def _act_post(p0, p1, d0, d1, un_in, un_out, Wa_in2, ba_in2, Wa_out2,
              ba_out2, hn, Wc, bc):
    return pl.pallas_call(
        _act_post_body,
        grid=(_NB,),
        in_specs=[_row_spec(_DT), _row_spec(_DT), _row_spec(1), _row_spec(1),
                  _row_spec(2), _row_spec(2),
                  _full_spec((HIDDEN, 2)), _full_spec((1, 2)),
                  _full_spec((HIDDEN, 2)), _full_spec((1, 2)),
                  _row_spec(ENV_DIM), _full_spec((ENV_DIM, ENV_DIM)),
                  _full_spec((1, ENV_DIM))],
        out_specs=[_row_spec(_DT), _row_spec(4), _row_spec(1)],
        out_shape=[jax.ShapeDtypeStruct((_NROW, _DT), jnp.float32),
                   jax.ShapeDtypeStruct((N, 4), jnp.float32),
                   jax.ShapeDtypeStruct((_NROW, 1), jnp.float32)],
    )(p0, p1, d0, d1, un_in, un_out, Wa_in2, ba_in2.reshape(1, -1), Wa_out2,
      ba_out2.reshape(1, -1), hn, Wc, bc.reshape(1, -1))


def _conv_post_body(s0_ref, s1_ref, w0_ref, w1_ref, hn_ref, act_ref, h_ref):
    s = s0_ref[...] + s1_ref[...]
    wdeg = jnp.maximum(w0_ref[...] + w1_ref[...], 1e-6)
    ki = act_ref[:, 0:1] + act_ref[:, 1:2]
    h_ref[...] = hn_ref[...] + ki * jax.nn.relu(s / wdeg)


def _conv_post(s0, s1, w0, w1, hn, act):
    return pl.pallas_call(
        _conv_post_body,
        grid=(_NB,),
        in_specs=[_row_spec(_DT), _row_spec(_DT), _row_spec(1), _row_spec(1),
                  _row_spec(ENV_DIM), _row_spec(4)],
        out_specs=_row_spec(ENV_DIM),
        out_shape=jax.ShapeDtypeStruct((N, ENV_DIM), jnp.float32),
    )(s0, s1, w0, w1, hn, act)


def _final_body(h_ref, g_ref, b_ref, batch_ref, wf_ref, bf_ref,
                pool_ref, res_ref):
    i = pl.program_id(0)

    @pl.when(i == 0)
    def _():
        pool_ref[...] = jnp.zeros((NUM_GRAPHS, ENV_DIM + 16), jnp.float32)

    rows = i * _R + lax.broadcasted_iota(jnp.int32, (_R, 1), 0)
    valid = rows < N
    hf = _ln(h_ref[...], g_ref[...], b_ref[...])
    hf = jnp.where(valid, hf, 0.0)
    gids = lax.broadcasted_iota(jnp.int32, (_R, NUM_GRAPHS), 1)
    onehot = jnp.where((batch_ref[...] == gids) & valid, 1.0, 0.0)
    hext = jnp.concatenate(
        [hf, valid.astype(jnp.float32), jnp.zeros((_R, 15), jnp.float32)],
        axis=1)
    pool_ref[...] += lax.dot_general(onehot, hext, (((0,), (0,)), ((), ())))

    @pl.when(i == _NB - 1)
    def _():
        p = pool_ref[...]
        pooled = p[:, :ENV_DIM] / jnp.maximum(p[:, ENV_DIM:ENV_DIM + 1], 1.0)
        res_ref[...] = jnp.dot(pooled, wf_ref[...]) + bf_ref[...]


def _final(h, ln_g, ln_b, batch2d, W_fin, b_fin):
    pooled, res = pl.pallas_call(
        _final_body,
        grid=(_NB,),
        in_specs=[_row_spec(ENV_DIM), _full_spec((1, ENV_DIM)),
                  _full_spec((1, ENV_DIM)), _row_spec(1),
                  _full_spec((ENV_DIM, OUT_DIM)), _full_spec((1, OUT_DIM))],
        out_specs=[_full_spec((NUM_GRAPHS, ENV_DIM + 16)),
                   _full_spec((NUM_GRAPHS, OUT_DIM))],
        out_shape=[jax.ShapeDtypeStruct((NUM_GRAPHS, ENV_DIM + 16),
                                        jnp.float32),
                   jax.ShapeDtypeStruct((NUM_GRAPHS, OUT_DIM), jnp.float32)],
    )(h, ln_g.reshape(1, -1), ln_b.reshape(1, -1), batch2d, W_fin,
      b_fin.reshape(1, -1))
    return res


def kernel(x, edge_index, pestat, batch, W_in, b_in, ln_g, ln_b, W_conv,
           b_conv, Wa_in1, ba_in1, Wa_in2, ba_in2, Wa_out1, ba_out1,
           Wa_out2, ba_out2, W_fin, b_fin):
    del pestat
    pad = N + (jnp.arange(_EPAD - E, dtype=jnp.int32) % (_NROW - N))
    u_p = jnp.concatenate([edge_index[0], pad])
    v_p = jnp.concatenate([edge_index[1], pad])
    Wa1 = jnp.concatenate([Wa_in1, Wa_out1], axis=1)
    b1 = jnp.concatenate([ba_in1, ba_out1])
    ones_k = jnp.ones((_NROW,), jnp.float32)

    uns = []
    for i in range(NUM_LAYERS):
        k = jax.random.fold_in(jax.random.key(42), i)
        uns.append(tuple(
            jax.random.uniform(jax.random.fold_in(k, j), (N, 2),
                               minval=1e-6, maxval=1.0 - 1e-6)
            for j in range(2)))

    h = _enc(x, W_in, b_in)
    pd = _sc_hist(ones_k, u_p, v_p)
    d0 = pd[0].reshape(_NROW, 1)
    d1 = pd[1].reshape(_NROW, 1)
    hist = jnp.zeros((N, HIST_DIM), jnp.float32)
    acts = []
    for i in range(NUM_LAYERS):
        hn, A = _build_a(h, hist, ln_g, ln_b, Wa1, b1)
        pz = _sc_pass(A, u_p, v_p)
        ctab, act, ko = _act_post(
            pz[0], pz[1], d0, d1,
            uns[i][0], uns[i][1], Wa_in2, ba_in2, Wa_out2, ba_out2, hn,
            W_conv[i], b_conv[i])
        s = _sc_pass(ctab, u_p, v_p)
        sw = _sc_hist(ko.reshape(_NROW), u_p, v_p)
        h = _conv_post(s[0], s[1], sw[0].reshape(_NROW, 1),
                       sw[1].reshape(_NROW, 1), hn, act)
        acts.append(act)
        if i < NUM_LAYERS - 1:
            hist = jnp.concatenate([hist[:, 4:], act], axis=1)

    result = _final(h, ln_g, ln_b, batch.reshape(N, 1), W_fin, b_fin)
    history = jnp.concatenate(
        [jnp.zeros((N, 4), x.dtype), acts[0], acts[1]], axis=1)
    return (result, -jnp.ones((NUM_LAYERS,), x.dtype), history)


# speedup vs baseline: 3.0909x; 1.0535x over previous
import functools

import jax
import jax.numpy as jnp
from jax import lax
from jax.experimental import pallas as pl
from jax.experimental.pallas import tpu as pltpu
from jax.experimental.pallas import tpu_sc as plsc

N = 10000
E = 320000
D_FEAT = 128
ENV_DIM = 128
HIDDEN = 64
NUM_LAYERS = 3
HIST_DIM = 12
NUM_GRAPHS = 64
OUT_DIM = 40
TAU = 0.5

_NC, _NS = 2, 16
_NW = _NC * _NS
_C = 128
_NCHUNK = 80
_EPW = _C * _NCHUNK
_EPAD = _EPW * _NW
_DT = 128
_NROW = 10240
_RPT = _NROW // _NS
_HR = _NROW // 128

_R = 512
_NB = _NROW // _R


def _sc_pass_body(table, u, v, out,
                  uall, v0, v1, rows0, rows1, acc,
                  sg0, sg1, ss0, ss1, si0, si1):
    cid = lax.axis_index("c")
    sid = lax.axis_index("s")
    wid = cid * _NS + sid
    base = wid * _EPW

    zero16 = jnp.zeros((16,), jnp.float32)

    def _zr(i, carry):
        for j in range(_DT // 16):
            rows0[i, pl.ds(j * 16, 16)] = zero16
        return carry
    lax.fori_loop(0, _C, _zr, 0)

    pltpu.sync_copy(u.at[pl.ds(base, _EPW)], uall)
    for k in range(_RPT // _C):
        pltpu.sync_copy(rows0, acc.at[pl.ds(sid * _RPT + k * _C, _C)])

    pltpu.async_copy(v.at[pl.ds(base, _C)], v0, si0)
    pltpu.async_copy(table.at[uall.at[pl.ds(0, _C)]], rows0, sg0)
    plsc.subcore_barrier()

    bufs = ((v0, rows0, sg0, ss0, si0), (v1, rows1, sg1, ss1, si1))

    def _drain_rows(sem, rbuf):
        pltpu.make_async_copy(table.at[pl.ds(0, _C)], rbuf, sem).wait()

    def _pair(gg, carry):
        for b in range(2):
            c = gg * 2 + b
            vc, rc, sgc, ssc, sic = bufs[b]
            vn, rn, sgn, ssn, sin = bufs[1 - b]

            @pl.when(c >= 1)
            def _():
                _drain_rows(ssn, rn)

            @pl.when(c + 1 < _NCHUNK)
            def _():
                pltpu.async_copy(v.at[pl.ds(base + (c + 1) * _C, _C)],
                                 vn, sin)
                pltpu.async_copy(
                    table.at[uall.at[pl.ds((c + 1) * _C, _C)]], rn, sgn)

            pltpu.make_async_copy(v.at[pl.ds(0, _C)], vc, sic).wait()
            _drain_rows(sgc, rc)
            pltpu.async_copy(rc, acc.at[vc], ssc, add=True)
        return carry
    lax.fori_loop(0, _NCHUNK // 2, _pair, 0)

    _drain_rows(ss1, rows1)
    plsc.subcore_barrier()

    pltpu.sync_copy(acc.at[pl.ds(sid * _RPT, _RPT)],
                    out.at[cid, pl.ds(sid * _RPT, _RPT)])


@functools.cache
def _get_sc_pass():
    return pl.kernel(
        _sc_pass_body,
        out_type=jax.ShapeDtypeStruct((_NC, _NROW, _DT), jnp.float32),
        mesh=plsc.VectorSubcoreMesh(core_axis_name="c", subcore_axis_name="s",
                                    num_cores=_NC, num_subcores=_NS),
        compiler_params=pltpu.CompilerParams(needs_layout_passes=False),
        scratch_types=[
            pltpu.VMEM((_EPW,), jnp.int32),
            pltpu.VMEM((_C,), jnp.int32),
            pltpu.VMEM((_C,), jnp.int32),
            pltpu.VMEM((_C, _DT), jnp.float32),
            pltpu.VMEM((_C, _DT), jnp.float32),
            pltpu.VMEM_SHARED((_NROW, _DT), jnp.float32),
            pltpu.SemaphoreType.DMA,
            pltpu.SemaphoreType.DMA,
            pltpu.SemaphoreType.DMA,
            pltpu.SemaphoreType.DMA,
            pltpu.SemaphoreType.DMA,
            pltpu.SemaphoreType.DMA,
        ],
    )


def _sc_pass(table, u, v):
    return _get_sc_pass()(table, u, v)


def _sc_hist_body(kvec, u, v, out_h, kv_v, u_all, v_all, hist, idx_h, acc_h):
    cid = lax.axis_index("c")
    sid = lax.axis_index("s")
    wid = cid * _NS + sid

    zero16 = jnp.zeros((16,), jnp.float32)

    def _zh(i, carry):
        for j in range(128 // 16):
            hist[i, pl.ds(j * 16, 16)] = zero16
        return carry
    lax.fori_loop(0, _HR, _zh, 0)

    iota16 = lax.iota(jnp.int32, 16)
    for j in range(_HR // 16):
        idx_h[pl.ds(j * 16, 16)] = iota16 + j * 16

    @pl.when(sid < _HR // 8)
    def _():
        pltpu.sync_copy(hist.at[pl.ds(0, 8)], acc_h.at[pl.ds(sid * 8, 8)])

    pltpu.sync_copy(kvec, kv_v)
    pltpu.sync_copy(u.at[pl.ds(wid * _EPW, _EPW)], u_all)
    pltpu.sync_copy(v.at[pl.ds(wid * _EPW, _EPW)], v_all)
    plsc.subcore_barrier()

    def _body(i, carry):
        u16 = u_all[pl.ds(i * 16, 16)]
        v16 = v_all[pl.ds(i * 16, 16)]
        kv16 = plsc.load_gather(kv_v, [u16])
        plsc.addupdate_scatter(hist, [v16 >> 7, v16 & 127], kv16)
        return carry
    lax.fori_loop(0, _EPW // 16, _body, 0)

    pltpu.sync_copy(hist, acc_h.at[idx_h], add=True)
    plsc.subcore_barrier()

    @pl.when(sid < _HR // 8)
    def _():
        pltpu.sync_copy(acc_h.at[pl.ds(sid * 8, 8)],
                        out_h.at[cid, pl.ds(sid * 8, 8)])


@functools.cache
def _get_sc_hist():
    return pl.kernel(
        _sc_hist_body,
        out_type=jax.ShapeDtypeStruct((_NC, _HR, 128), jnp.float32),
        mesh=plsc.VectorSubcoreMesh(core_axis_name="c", subcore_axis_name="s",
                                    num_cores=_NC, num_subcores=_NS),
        compiler_params=pltpu.CompilerParams(needs_layout_passes=False),
        scratch_types=[
            pltpu.VMEM((_NROW,), jnp.float32),
            pltpu.VMEM((_EPW,), jnp.int32),
            pltpu.VMEM((_EPW,), jnp.int32),
            pltpu.VMEM((_HR, 128), jnp.float32),
            pltpu.VMEM((_HR,), jnp.int32),
            pltpu.VMEM_SHARED((_HR, 128), jnp.float32),
        ],
    )


def _sc_hist(kvec, u, v):
    return _get_sc_hist()(kvec, u, v)


def _row_spec(w):
    return pl.BlockSpec((_R, w), lambda i: (i, 0))


def _full_spec(shape):
    nd = len(shape)
    return pl.BlockSpec(shape, lambda i, _n=nd: (0,) * _n)


def _ln(hb, g, b):
    mu = jnp.mean(hb, axis=-1, keepdims=True)
    var = jnp.mean((hb - mu) ** 2, axis=-1, keepdims=True)
    return (hb - mu) / jnp.sqrt(var + 1e-5) * g + b


def _hist_spec():
    return pl.BlockSpec((_R, 1), lambda i: (i, 0))


def _enc_a_body(x_ref, w_ref, b_ref, g_ref, lb_ref, w1_ref, b1_ref,
                hn_ref, a_ref):
    h = jax.nn.relu(jnp.dot(x_ref[...], w_ref[...]) + b_ref[...])
    hn = _ln(h, g_ref[...], lb_ref[...])
    hn_ref[...] = hn
    a_ref[...] = jax.nn.relu(jnp.dot(hn, w1_ref[...][:ENV_DIM])
                             + b1_ref[...])


def _enc_a(x, W_in, b_in, ln_g, ln_b, Wa1, b1):
    return pl.pallas_call(
        _enc_a_body,
        grid=(_NB,),
        in_specs=[_row_spec(D_FEAT), _full_spec((D_FEAT, ENV_DIM)),
                  _full_spec((1, ENV_DIM)), _full_spec((1, ENV_DIM)),
                  _full_spec((1, ENV_DIM)),
                  _full_spec((ENV_DIM + HIST_DIM, 2 * HIDDEN)),
                  _full_spec((1, 2 * HIDDEN))],
        out_specs=[_row_spec(ENV_DIM), _row_spec(_DT)],
        out_shape=[jax.ShapeDtypeStruct((N, ENV_DIM), jnp.float32),
                   jax.ShapeDtypeStruct((_NROW, _DT), jnp.float32)],
    )(x, W_in, b_in.reshape(1, -1), ln_g.reshape(1, -1),
      ln_b.reshape(1, -1), Wa1, b1.reshape(1, -1))


def _keep(agg, W2, b2, un):
    logits = jnp.dot(agg, W2) + b2
    g = -jnp.log(-jnp.log(un))
    t = (logits + g) / TAU
    m = jnp.max(t, axis=-1, keepdims=True)
    e = jnp.exp(t - m)
    y = e / jnp.sum(e, axis=-1, keepdims=True)
    return jnp.where(y[:, 0:1] >= y[:, 1:2], 1.0, 0.0)


def _act_post_body(p0_ref, p1_ref, d0_ref, d1_ref, uni_ref, uno_ref,
                   wi2_ref, bi2_ref, wo2_ref, bo2_ref, hn_ref, wc_ref,
                   bc_ref, c_ref, act_ref, ko_ref):
    p = p0_ref[...] + p1_ref[...]
    deg = jnp.maximum(d0_ref[...] + d1_ref[...], 1.0)
    ki = _keep(p[:, :HIDDEN] / deg, wi2_ref[...], bi2_ref[...], uni_ref[...])
    ko = _keep(p[:, HIDDEN:ENV_DIM] / deg, wo2_ref[...], bo2_ref[...],
               uno_ref[...])
    m = jnp.dot(hn_ref[...], wc_ref[...]) + bc_ref[...]
    c_ref[...] = ko * m
    ko_ref[...] = ko
    act_ref[...] = jnp.concatenate(
        [ki * ko, ki * (1.0 - ko), (1.0 - ki) * ko, (1.0 - ki) * (1.0 - ko)],
        axis=1)


def _act_post(p0, p1, d0, d1, un_in, un_out, Wa_in2, ba_in2, Wa_out2,
              ba_out2, hn, Wc, bc):
    return pl.pallas_call(
        _act_post_body,
        grid=(_NB,),
        in_specs=[_row_spec(_DT), _row_spec(_DT), _hist_spec(), _hist_spec(),
                  _row_spec(2), _row_spec(2),
                  _full_spec((HIDDEN, 2)), _full_spec((1, 2)),
                  _full_spec((HIDDEN, 2)), _full_spec((1, 2)),
                  _row_spec(ENV_DIM), _full_spec((ENV_DIM, ENV_DIM)),
                  _full_spec((1, ENV_DIM))],
        out_specs=[_row_spec(_DT), _row_spec(4), _row_spec(1)],
        out_shape=[jax.ShapeDtypeStruct((_NROW, _DT), jnp.float32),
                   jax.ShapeDtypeStruct((N, 4), jnp.float32),
                   jax.ShapeDtypeStruct((_NROW, 1), jnp.float32)],
    )(p0, p1, d0, d1, un_in, un_out, Wa_in2, ba_in2.reshape(1, -1), Wa_out2,
      ba_out2.reshape(1, -1), hn, Wc, bc.reshape(1, -1))


def _new_h(s0_ref, s1_ref, w0_ref, w1_ref, hn_ref, act_ref):
    s = s0_ref[...] + s1_ref[...]
    wdeg = jnp.maximum(w0_ref[...] + w1_ref[...], 1e-6)
    ki = act_ref[:, 0:1] + act_ref[:, 1:2]
    return hn_ref[...] + ki * jax.nn.relu(s / wdeg)


def _conv_a_body(s0_ref, s1_ref, w0_ref, w1_ref, hn_ref, act_ref,
                 hist_ref, g_ref, lb_ref, w1a_ref, b1_ref,
                 hn2_ref, a_ref):
    h = _new_h(s0_ref, s1_ref, w0_ref, w1_ref, hn_ref, act_ref)
    hn = _ln(h, g_ref[...], lb_ref[...])
    hn2_ref[...] = hn
    w1 = w1a_ref[...]
    z = jnp.dot(hn, w1[:ENV_DIM]) + jnp.dot(hist_ref[...], w1[ENV_DIM:])
    a_ref[...] = jax.nn.relu(z + b1_ref[...])


def _conv_a(s0, s1, w0, w1, hn, act, hist, ln_g, ln_b, Wa1, b1):
    return pl.pallas_call(
        _conv_a_body,
        grid=(_NB,),
        in_specs=[_row_spec(_DT), _row_spec(_DT), _hist_spec(), _hist_spec(),
                  _row_spec(ENV_DIM), _row_spec(4), _row_spec(HIST_DIM),
                  _full_spec((1, ENV_DIM)), _full_spec((1, ENV_DIM)),
                  _full_spec((ENV_DIM + HIST_DIM, 2 * HIDDEN)),
                  _full_spec((1, 2 * HIDDEN))],
        out_specs=[_row_spec(ENV_DIM), _row_spec(_DT)],
        out_shape=[jax.ShapeDtypeStruct((N, ENV_DIM), jnp.float32),
                   jax.ShapeDtypeStruct((_NROW, _DT), jnp.float32)],
    )(s0, s1, w0, w1, hn, act, hist, ln_g.reshape(1, -1),
      ln_b.reshape(1, -1), Wa1, b1.reshape(1, -1))


def _conv_final_body(s0_ref, s1_ref, w0_ref, w1_ref, hn_ref, act_ref,
                     g_ref, lb_ref, batch_ref, wf_ref, bf_ref,
                     pool_ref, res_ref):
    i = pl.program_id(0)

    @pl.when(i == 0)
    def _():
        pool_ref[...] = jnp.zeros((NUM_GRAPHS, ENV_DIM + 16), jnp.float32)

    h = _new_h(s0_ref, s1_ref, w0_ref, w1_ref, hn_ref, act_ref)
    rows = i * _R + lax.broadcasted_iota(jnp.int32, (_R, 1), 0)
    valid = rows < N
    hf = _ln(h, g_ref[...], lb_ref[...])
    hf = jnp.where(valid, hf, 0.0)
    gids = lax.broadcasted_iota(jnp.int32, (_R, NUM_GRAPHS), 1)
    onehot = jnp.where((batch_ref[...] == gids) & valid, 1.0, 0.0)
    hext = jnp.concatenate(
        [hf, valid.astype(jnp.float32), jnp.zeros((_R, 15), jnp.float32)],
        axis=1)
    pool_ref[...] += lax.dot_general(onehot, hext, (((0,), (0,)), ((), ())))

    @pl.when(i == _NB - 1)
    def _():
        p = pool_ref[...]
        pooled = p[:, :ENV_DIM] / jnp.maximum(p[:, ENV_DIM:ENV_DIM + 1], 1.0)
        res_ref[...] = jnp.dot(pooled, wf_ref[...]) + bf_ref[...]


def _conv_final(s0, s1, w0, w1, hn, act, ln_g, ln_b, batch2d, W_fin, b_fin):
    pooled, res = pl.pallas_call(
        _conv_final_body,
        grid=(_NB,),
        in_specs=[_row_spec(_DT), _row_spec(_DT), _hist_spec(), _hist_spec(),
                  _row_spec(ENV_DIM), _row_spec(4),
                  _full_spec((1, ENV_DIM)), _full_spec((1, ENV_DIM)),
                  _row_spec(1),
                  _full_spec((ENV_DIM, OUT_DIM)), _full_spec((1, OUT_DIM))],
        out_specs=[_full_spec((NUM_GRAPHS, ENV_DIM + 16)),
                   _full_spec((NUM_GRAPHS, OUT_DIM))],
        out_shape=[jax.ShapeDtypeStruct((NUM_GRAPHS, ENV_DIM + 16),
                                        jnp.float32),
                   jax.ShapeDtypeStruct((NUM_GRAPHS, OUT_DIM), jnp.float32)],
    )(s0, s1, w0, w1, hn, act, ln_g.reshape(1, -1), ln_b.reshape(1, -1),
      batch2d, W_fin, b_fin.reshape(1, -1))
    return res


def kernel(x, edge_index, pestat, batch, W_in, b_in, ln_g, ln_b, W_conv,
           b_conv, Wa_in1, ba_in1, Wa_in2, ba_in2, Wa_out1, ba_out1,
           Wa_out2, ba_out2, W_fin, b_fin):
    del pestat
    pad = N + (jnp.arange(_EPAD - E, dtype=jnp.int32) % (_NROW - N))
    u_p = jnp.concatenate([edge_index[0], pad])
    v_p = jnp.concatenate([edge_index[1], pad])
    Wa1 = jnp.concatenate([Wa_in1, Wa_out1], axis=1)
    b1 = jnp.concatenate([ba_in1, ba_out1])
    ones_k = jnp.ones((_NROW,), jnp.float32)

    uns = []
    for i in range(NUM_LAYERS):
        k = jax.random.fold_in(jax.random.key(42), i)
        uns.append(tuple(
            jax.random.uniform(jax.random.fold_in(k, j), (N, 2),
                               minval=1e-6, maxval=1.0 - 1e-6)
            for j in range(2)))

    pd = _sc_hist(ones_k, u_p, v_p)
    d0 = pd[0].reshape(_NROW, 1)
    d1 = pd[1].reshape(_NROW, 1)
    hn, A = _enc_a(x, W_in, b_in, ln_g, ln_b, Wa1, b1)
    zeros4 = jnp.zeros((N, 4), jnp.float32)
    acts = []
    result = None
    for i in range(NUM_LAYERS):
        pz = _sc_pass(A, u_p, v_p)
        ctab, act, ko = _act_post(
            pz[0], pz[1], d0, d1,
            uns[i][0], uns[i][1], Wa_in2, ba_in2, Wa_out2, ba_out2, hn,
            W_conv[i], b_conv[i])
        acts.append(act)
        s = _sc_pass(ctab, u_p, v_p)
        swp = _sc_hist(ko.reshape(_NROW), u_p, v_p)
        sw = (swp[0].reshape(_NROW, 1), swp[1].reshape(_NROW, 1))
        if i < NUM_LAYERS - 1:
            hist = jnp.concatenate(
                [zeros4] * (NUM_LAYERS - 1 - i) + acts[max(0, i - 1):],
                axis=1)
            hn, A = _conv_a(s[0], s[1], sw[0], sw[1], hn, act, hist,
                            ln_g, ln_b, Wa1, b1)
        else:
            result = _conv_final(s[0], s[1], sw[0], sw[1], hn, act,
                                 ln_g, ln_b, batch.reshape(N, 1),
                                 W_fin, b_fin)
    history = jnp.concatenate(
        [jnp.zeros((N, 4), x.dtype), acts[0], acts[1]], axis=1)
    return (result, -jnp.ones((NUM_LAYERS,), x.dtype), history)


# speedup vs baseline: 3.1980x; 1.0347x over previous
import functools

import jax
import jax.numpy as jnp
from jax import lax
from jax.experimental import pallas as pl
from jax.experimental.pallas import tpu as pltpu
from jax.experimental.pallas import tpu_sc as plsc

N = 10000
E = 320000
D_FEAT = 128
ENV_DIM = 128
HIDDEN = 64
NUM_LAYERS = 3
HIST_DIM = 12
NUM_GRAPHS = 64
OUT_DIM = 40
TAU = 0.5

_NC, _NS = 2, 16
_NW = _NC * _NS
_C = 128
_NCHUNK = 80
_EPW = _C * _NCHUNK
_EPAD = _EPW * _NW
_DT = 128
_NROW = 10240
_RPT = _NROW // _NS
_HR = _NROW // 128

_R = 512
_NB = _NROW // _R


def _sc_pass_body(table, u, v, out,
                  uall, v0, v1, rows0, rows1, acc,
                  sg0, sg1, ss0, ss1, si0, si1):
    cid = lax.axis_index("c")
    sid = lax.axis_index("s")
    wid = cid * _NS + sid
    base = wid * _EPW

    zero16 = jnp.zeros((16,), jnp.float32)

    def _zr(i, carry):
        for j in range(_DT // 16):
            rows0[i, pl.ds(j * 16, 16)] = zero16
        return carry
    lax.fori_loop(0, _C, _zr, 0)

    pltpu.sync_copy(u.at[pl.ds(base, _EPW)], uall)
    for k in range(_RPT // _C):
        pltpu.sync_copy(rows0, acc.at[pl.ds(sid * _RPT + k * _C, _C)])

    pltpu.async_copy(v.at[pl.ds(base, _C)], v0, si0)
    pltpu.async_copy(table.at[uall.at[pl.ds(0, _C)]], rows0, sg0)
    plsc.subcore_barrier()

    bufs = ((v0, rows0, sg0, ss0, si0), (v1, rows1, sg1, ss1, si1))

    def _drain_rows(sem, rbuf):
        pltpu.make_async_copy(table.at[pl.ds(0, _C)], rbuf, sem).wait()

    def _pair(gg, carry):
        for b in range(2):
            c = gg * 2 + b
            vc, rc, sgc, ssc, sic = bufs[b]
            vn, rn, sgn, ssn, sin = bufs[1 - b]

            @pl.when(c >= 1)
            def _():
                _drain_rows(ssn, rn)

            @pl.when(c + 1 < _NCHUNK)
            def _():
                pltpu.async_copy(v.at[pl.ds(base + (c + 1) * _C, _C)],
                                 vn, sin)
                pltpu.async_copy(
                    table.at[uall.at[pl.ds((c + 1) * _C, _C)]], rn, sgn)

            pltpu.make_async_copy(v.at[pl.ds(0, _C)], vc, sic).wait()
            _drain_rows(sgc, rc)
            pltpu.async_copy(rc, acc.at[vc], ssc, add=True)
        return carry
    lax.fori_loop(0, _NCHUNK // 2, _pair, 0)

    _drain_rows(ss1, rows1)
    plsc.subcore_barrier()

    pltpu.sync_copy(acc.at[pl.ds(sid * _RPT, _RPT)],
                    out.at[cid, pl.ds(sid * _RPT, _RPT)])


@functools.cache
def _get_sc_pass():
    return pl.kernel(
        _sc_pass_body,
        out_type=jax.ShapeDtypeStruct((_NC, _NROW, _DT), jnp.float32),
        mesh=plsc.VectorSubcoreMesh(core_axis_name="c", subcore_axis_name="s",
                                    num_cores=_NC, num_subcores=_NS),
        compiler_params=pltpu.CompilerParams(needs_layout_passes=False),
        scratch_types=[
            pltpu.VMEM((_EPW,), jnp.int32),
            pltpu.VMEM((_C,), jnp.int32),
            pltpu.VMEM((_C,), jnp.int32),
            pltpu.VMEM((_C, _DT), jnp.float32),
            pltpu.VMEM((_C, _DT), jnp.float32),
            pltpu.VMEM_SHARED((_NROW, _DT), jnp.float32),
            pltpu.SemaphoreType.DMA,
            pltpu.SemaphoreType.DMA,
            pltpu.SemaphoreType.DMA,
            pltpu.SemaphoreType.DMA,
            pltpu.SemaphoreType.DMA,
            pltpu.SemaphoreType.DMA,
        ],
    )


def _sc_pass(table, u, v):
    return _get_sc_pass()(table, u, v)


def _sc_hist_body(kvec, u, v, out_h, kv_v, u_all, v_all, hist, idx_h, acc_h):
    cid = lax.axis_index("c")
    sid = lax.axis_index("s")
    wid = cid * _NS + sid

    zero16 = jnp.zeros((16,), jnp.float32)

    def _zh(i, carry):
        for j in range(128 // 16):
            hist[i, pl.ds(j * 16, 16)] = zero16
        return carry
    lax.fori_loop(0, _HR, _zh, 0)

    iota16 = lax.iota(jnp.int32, 16)
    for j in range(_HR // 16):
        idx_h[pl.ds(j * 16, 16)] = iota16 + j * 16

    @pl.when(sid < _HR // 8)
    def _():
        pltpu.sync_copy(hist.at[pl.ds(0, 8)], acc_h.at[pl.ds(sid * 8, 8)])

    pltpu.sync_copy(kvec, kv_v)
    pltpu.sync_copy(u.at[pl.ds(wid * _EPW, _EPW)], u_all)
    pltpu.sync_copy(v.at[pl.ds(wid * _EPW, _EPW)], v_all)
    plsc.subcore_barrier()

    def _body(i, carry):
        u16 = u_all[pl.ds(i * 16, 16)]
        v16 = v_all[pl.ds(i * 16, 16)]
        kv16 = plsc.load_gather(kv_v, [u16])
        plsc.addupdate_scatter(hist, [v16 >> 7, v16 & 127], kv16)
        return carry
    lax.fori_loop(0, _EPW // 16, _body, 0)

    pltpu.sync_copy(hist, acc_h.at[idx_h], add=True)
    plsc.subcore_barrier()

    @pl.when(sid < _HR // 8)
    def _():
        pltpu.sync_copy(acc_h.at[pl.ds(sid * 8, 8)],
                        out_h.at[cid, pl.ds(sid * 8, 8)])


@functools.cache
def _get_sc_hist():
    return pl.kernel(
        _sc_hist_body,
        out_type=jax.ShapeDtypeStruct((_NC, _HR, 128), jnp.float32),
        mesh=plsc.VectorSubcoreMesh(core_axis_name="c", subcore_axis_name="s",
                                    num_cores=_NC, num_subcores=_NS),
        compiler_params=pltpu.CompilerParams(needs_layout_passes=False),
        scratch_types=[
            pltpu.VMEM((_NROW,), jnp.float32),
            pltpu.VMEM((_EPW,), jnp.int32),
            pltpu.VMEM((_EPW,), jnp.int32),
            pltpu.VMEM((_HR, 128), jnp.float32),
            pltpu.VMEM((_HR,), jnp.int32),
            pltpu.VMEM_SHARED((_HR, 128), jnp.float32),
        ],
    )


def _sc_hist(kvec, u, v):
    return _get_sc_hist()(kvec, u, v)


def _row_spec(w):
    return pl.BlockSpec((_R, w), lambda i: (i, 0))


def _full_spec(shape):
    nd = len(shape)
    return pl.BlockSpec(shape, lambda i, _n=nd: (0,) * _n)


def _ln(hb, g, b):
    mu = jnp.mean(hb, axis=-1, keepdims=True)
    var = jnp.mean((hb - mu) ** 2, axis=-1, keepdims=True)
    return (hb - mu) / jnp.sqrt(var + 1e-5) * g + b


def _pair_spec(w):
    return pl.BlockSpec((2, _R, w), lambda i: (0, i, 0))


def _enc_a_body(x_ref, w_ref, b_ref, g_ref, lb_ref, w1_ref, b1_ref,
                hn_ref, a_ref):
    h = jax.nn.relu(jnp.dot(x_ref[...], w_ref[...]) + b_ref[...])
    hn = _ln(h, g_ref[...], lb_ref[...])
    hn_ref[...] = hn
    a_ref[...] = jax.nn.relu(jnp.dot(hn, w1_ref[...][:ENV_DIM])
                             + b1_ref[...])


def _enc_a(x, W_in, b_in, ln_g, ln_b, Wa1, b1):
    return pl.pallas_call(
        _enc_a_body,
        grid=(_NB,),
        in_specs=[_row_spec(D_FEAT), _full_spec((D_FEAT, ENV_DIM)),
                  _full_spec((1, ENV_DIM)), _full_spec((1, ENV_DIM)),
                  _full_spec((1, ENV_DIM)),
                  _full_spec((ENV_DIM + HIST_DIM, 2 * HIDDEN)),
                  _full_spec((1, 2 * HIDDEN))],
        out_specs=[_row_spec(ENV_DIM), _row_spec(_DT)],
        out_shape=[jax.ShapeDtypeStruct((N, ENV_DIM), jnp.float32),
                   jax.ShapeDtypeStruct((_NROW, _DT), jnp.float32)],
    )(x, W_in, b_in.reshape(1, -1), ln_g.reshape(1, -1),
      ln_b.reshape(1, -1), Wa1, b1.reshape(1, -1))


def _keep(agg, W2, b2, un):
    logits = jnp.dot(agg, W2) + b2
    g = -jnp.log(-jnp.log(un))
    t = (logits + g) / TAU
    m = jnp.max(t, axis=-1, keepdims=True)
    e = jnp.exp(t - m)
    y = e / jnp.sum(e, axis=-1, keepdims=True)
    return jnp.where(y[:, 0:1] >= y[:, 1:2], 1.0, 0.0)


def _act_post_body(p_ref, d_ref, uni_ref, uno_ref,
                   wi2_ref, bi2_ref, wo2_ref, bo2_ref, hn_ref, wc_ref,
                   bc_ref, c_ref, act_ref, ko_ref):
    p = p_ref[0] + p_ref[1]
    deg = jnp.maximum(d_ref[0] + d_ref[1], 1.0)
    ki = _keep(p[:, :HIDDEN] / deg, wi2_ref[...], bi2_ref[...], uni_ref[...])
    ko = _keep(p[:, HIDDEN:ENV_DIM] / deg, wo2_ref[...], bo2_ref[...],
               uno_ref[...])
    m = jnp.dot(hn_ref[...], wc_ref[...]) + bc_ref[...]
    c_ref[...] = ko * m
    ko_ref[...] = ko
    act_ref[...] = jnp.concatenate(
        [ki * ko, ki * (1.0 - ko), (1.0 - ki) * ko, (1.0 - ki) * (1.0 - ko)],
        axis=1)


def _act_post(pz, dg, un_in, un_out, Wa_in2, ba_in2, Wa_out2,
              ba_out2, hn, Wc, bc):
    return pl.pallas_call(
        _act_post_body,
        grid=(_NB,),
        in_specs=[_pair_spec(_DT), _pair_spec(1),
                  _row_spec(2), _row_spec(2),
                  _full_spec((HIDDEN, 2)), _full_spec((1, 2)),
                  _full_spec((HIDDEN, 2)), _full_spec((1, 2)),
                  _row_spec(ENV_DIM), _full_spec((ENV_DIM, ENV_DIM)),
                  _full_spec((1, ENV_DIM))],
        out_specs=[_row_spec(_DT), _row_spec(4), _row_spec(1)],
        out_shape=[jax.ShapeDtypeStruct((_NROW, _DT), jnp.float32),
                   jax.ShapeDtypeStruct((N, 4), jnp.float32),
                   jax.ShapeDtypeStruct((_NROW, 1), jnp.float32)],
    )(pz, dg, un_in, un_out, Wa_in2, ba_in2.reshape(1, -1), Wa_out2,
      ba_out2.reshape(1, -1), hn, Wc, bc.reshape(1, -1))


def _new_h(s_ref, w_ref, hn_ref, act_ref):
    s = s_ref[0] + s_ref[1]
    wdeg = jnp.maximum(w_ref[0] + w_ref[1], 1e-6)
    ki = act_ref[:, 0:1] + act_ref[:, 1:2]
    return hn_ref[...] + ki * jax.nn.relu(s / wdeg)


def _conv_a_body(s_ref, w_ref, hn_ref, act_ref,
                 hist_ref, g_ref, lb_ref, w1a_ref, b1_ref,
                 hn2_ref, a_ref):
    h = _new_h(s_ref, w_ref, hn_ref, act_ref)
    hn = _ln(h, g_ref[...], lb_ref[...])
    hn2_ref[...] = hn
    w1 = w1a_ref[...]
    z = jnp.dot(hn, w1[:ENV_DIM]) + jnp.dot(hist_ref[...], w1[ENV_DIM:])
    a_ref[...] = jax.nn.relu(z + b1_ref[...])


def _conv_a(s, w, hn, act, hist, ln_g, ln_b, Wa1, b1):
    return pl.pallas_call(
        _conv_a_body,
        grid=(_NB,),
        in_specs=[_pair_spec(_DT), _pair_spec(1),
                  _row_spec(ENV_DIM), _row_spec(4), _row_spec(HIST_DIM),
                  _full_spec((1, ENV_DIM)), _full_spec((1, ENV_DIM)),
                  _full_spec((ENV_DIM + HIST_DIM, 2 * HIDDEN)),
                  _full_spec((1, 2 * HIDDEN))],
        out_specs=[_row_spec(ENV_DIM), _row_spec(_DT)],
        out_shape=[jax.ShapeDtypeStruct((N, ENV_DIM), jnp.float32),
                   jax.ShapeDtypeStruct((_NROW, _DT), jnp.float32)],
    )(s, w, hn, act, hist, ln_g.reshape(1, -1),
      ln_b.reshape(1, -1), Wa1, b1.reshape(1, -1))


def _conv_final_body(s_ref, w_ref, hn_ref, act_ref,
                     g_ref, lb_ref, batch_ref, wf_ref, bf_ref,
                     pool_ref, res_ref):
    i = pl.program_id(0)

    @pl.when(i == 0)
    def _():
        pool_ref[...] = jnp.zeros((NUM_GRAPHS, ENV_DIM + 16), jnp.float32)

    h = _new_h(s_ref, w_ref, hn_ref, act_ref)
    rows = i * _R + lax.broadcasted_iota(jnp.int32, (_R, 1), 0)
    valid = rows < N
    hf = _ln(h, g_ref[...], lb_ref[...])
    hf = jnp.where(valid, hf, 0.0)
    gids = lax.broadcasted_iota(jnp.int32, (_R, NUM_GRAPHS), 1)
    onehot = jnp.where((batch_ref[...] == gids) & valid, 1.0, 0.0)
    hext = jnp.concatenate(
        [hf, valid.astype(jnp.float32), jnp.zeros((_R, 15), jnp.float32)],
        axis=1)
    pool_ref[...] += lax.dot_general(onehot, hext, (((0,), (0,)), ((), ())))

    @pl.when(i == _NB - 1)
    def _():
        p = pool_ref[...]
        pooled = p[:, :ENV_DIM] / jnp.maximum(p[:, ENV_DIM:ENV_DIM + 1], 1.0)
        res_ref[...] = jnp.dot(pooled, wf_ref[...]) + bf_ref[...]


def _conv_final(s, w, hn, act, ln_g, ln_b, batch2d, W_fin, b_fin):
    pooled, res = pl.pallas_call(
        _conv_final_body,
        grid=(_NB,),
        in_specs=[_pair_spec(_DT), _pair_spec(1),
                  _row_spec(ENV_DIM), _row_spec(4),
                  _full_spec((1, ENV_DIM)), _full_spec((1, ENV_DIM)),
                  _row_spec(1),
                  _full_spec((ENV_DIM, OUT_DIM)), _full_spec((1, OUT_DIM))],
        out_specs=[_full_spec((NUM_GRAPHS, ENV_DIM + 16)),
                   _full_spec((NUM_GRAPHS, OUT_DIM))],
        out_shape=[jax.ShapeDtypeStruct((NUM_GRAPHS, ENV_DIM + 16),
                                        jnp.float32),
                   jax.ShapeDtypeStruct((NUM_GRAPHS, OUT_DIM), jnp.float32)],
    )(s, w, hn, act, ln_g.reshape(1, -1), ln_b.reshape(1, -1),
      batch2d, W_fin, b_fin.reshape(1, -1))
    return res


def kernel(x, edge_index, pestat, batch, W_in, b_in, ln_g, ln_b, W_conv,
           b_conv, Wa_in1, ba_in1, Wa_in2, ba_in2, Wa_out1, ba_out1,
           Wa_out2, ba_out2, W_fin, b_fin):
    del pestat
    pad = N + (jnp.arange(_EPAD - E, dtype=jnp.int32) % (_NROW - N))
    u_p = jnp.concatenate([edge_index[0], pad])
    v_p = jnp.concatenate([edge_index[1], pad])
    Wa1 = jnp.concatenate([Wa_in1, Wa_out1], axis=1)
    b1 = jnp.concatenate([ba_in1, ba_out1])
    ones_k = jnp.ones((_NROW,), jnp.float32)

    uns = []
    for i in range(NUM_LAYERS):
        k = jax.random.fold_in(jax.random.key(42), i)
        uns.append(tuple(
            jax.random.uniform(jax.random.fold_in(k, j), (N, 2),
                               minval=1e-6, maxval=1.0 - 1e-6)
            for j in range(2)))

    dg = _sc_hist(ones_k, u_p, v_p).reshape(2, _NROW, 1)
    hn, A = _enc_a(x, W_in, b_in, ln_g, ln_b, Wa1, b1)
    zeros4 = jnp.zeros((N, 4), jnp.float32)
    acts = []
    result = None
    for i in range(NUM_LAYERS):
        pz = _sc_pass(A, u_p, v_p)
        ctab, act, ko = _act_post(
            pz, dg,
            uns[i][0], uns[i][1], Wa_in2, ba_in2, Wa_out2, ba_out2, hn,
            W_conv[i], b_conv[i])
        acts.append(act)
        s = _sc_pass(ctab, u_p, v_p)
        sw = _sc_hist(ko.reshape(_NROW), u_p, v_p).reshape(2, _NROW, 1)
        if i < NUM_LAYERS - 1:
            hist = jnp.concatenate(
                [zeros4] * (NUM_LAYERS - 1 - i) + acts[max(0, i - 1):],
                axis=1)
            hn, A = _conv_a(s, sw, hn, act, hist,
                            ln_g, ln_b, Wa1, b1)
        else:
            result = _conv_final(s, sw, hn, act,
                                 ln_g, ln_b, batch.reshape(N, 1),
                                 W_fin, b_fin)
    history = jnp.concatenate(
        [jnp.zeros((N, 4), x.dtype), acts[0], acts[1]], axis=1)
    return (result, -jnp.ones((NUM_LAYERS,), x.dtype), history)
